# baseline (device time: 46564 ns/iter reference)
import os

import jax
import jax.numpy as jnp
from jax import lax
from jax.experimental import pallas as pl
from jax.experimental.pallas import tpu as pltpu

N_DEV = 4
B_LOC = 512
D = 256
H_LOC = 512
H = N_DEV * H_LOC
BF16 = jnp.bfloat16

def _r1a(l): return 4 * l + 0
def _r1b(l): return 4 * l + 1
def _r2a(l): return 4 * l + 2
def _r2b(l): return 4 * l + 3
O1A, O1B, O2A, O2B = 12, 13, 14, 15

_SKIP_COMM = bool(int(os.environ.get("SKIP_COMM", "0")))


def kernel(x, Win0, Wout0, Win1, Wout1, Win2, Wout2):
    def body(x_ref, win0_ref, wout0_ref, win1_ref, wout1_ref, win2_ref,
             wout2_ref, out_ref, winb_ref, woutb_ref, send_sems, recv_sems):
        j = lax.axis_index("i")
        pa = jnp.bitwise_xor(j, 1)
        pb = jnp.bitwise_xor(j, 3)

        barrier = pltpu.get_barrier_semaphore()
        for nbr in (pa, pb):
            pl.semaphore_signal(barrier, inc=1, device_id=(nbr,),
                                device_id_type=pl.DeviceIdType.MESH)
        pl.semaphore_wait(barrier, 2)

        pending = []

        class _Dummy:
            def wait_recv(self):
                pass

            def wait_send(self):
                pass

        def rcopy(ref_expr, sem_idx, target):
            if _SKIP_COMM:
                return _Dummy()
            r = pltpu.make_async_remote_copy(
                src_ref=ref_expr,
                dst_ref=ref_expr,
                send_sem=send_sems.at[sem_idx],
                recv_sem=recv_sems.at[sem_idx],
                device_id=(target,),
                device_id_type=pl.DeviceIdType.MESH,
            )
            r.start()
            pending.append(r)
            return r

        def cols(k, half):
            return pl.ds(pl.multiple_of(k * H_LOC + 256 * half, 256), 256)

        def win_half(k, l, half):
            return winb_ref.at[l, :, cols(k, half)]

        def wout_half(k, l, half):
            return woutb_ref.at[l, cols(k, half), :]

        def send_pair(k, l, half, sem_idx, target):
            return (rcopy(win_half(k, l, half), sem_idx, target),
                    rcopy(wout_half(k, l, half), sem_idx, target))

        in_refs = ((win0_ref, wout0_ref), (win1_ref, wout1_ref),
                   (win2_ref, wout2_ref))

        r2_descs = {l: [] for l in range(3)}
        for l, (win, wout) in enumerate(in_refs):
            myblk = pl.ds(pl.multiple_of(j * H_LOC, H_LOC), H_LOC)
            winb_ref[l, :, myblk] = win[:, :].astype(BF16)
            woutb_ref[l, myblk, :] = wout[:, :].astype(BF16)
            r1a = send_pair(j, l, 0, _r1a(l), pa)
            r1b = send_pair(j, l, 1, _r1b(l), pb)
            r2_descs[l] += send_pair(j, l, 0, _r2b(l), pb)
            r2_descs[l] += send_pair(j, l, 1, _r2a(l), pa)
            for r in r1a:
                r.wait_recv()
            r2_descs[l] += send_pair(pa, l, 0, _r2b(l), pb)
            for r in r1b:
                r.wait_recv()
            r2_descs[l] += send_pair(pb, l, 1, _r2a(l), pa)

        cur = x_ref[:, :].astype(BF16)
        for l in range(3):
            for r in r2_descs[l]:
                r.wait_recv()
            h = jnp.maximum(
                jnp.dot(cur, winb_ref[l, :, :],
                        preferred_element_type=jnp.float32),
                0.0,
            ).astype(BF16)
            cur = jnp.dot(h, woutb_ref[l, :, :],
                          preferred_element_type=jnp.float32).astype(BF16)
        out_ref[pl.ds(pl.multiple_of(j * B_LOC, B_LOC), B_LOC), :] = cur

        def ostrip(k, half):
            return out_ref.at[
                pl.ds(pl.multiple_of(k * B_LOC + 256 * half, 256), 256), :]

        o1a = rcopy(ostrip(j, 0), O1A, pa)
        o1b = rcopy(ostrip(j, 1), O1B, pb)
        o2 = [rcopy(ostrip(j, 0), O2B, pb),
              rcopy(ostrip(j, 1), O2A, pa)]
        o1a.wait_recv()
        o2.append(rcopy(ostrip(pa, 0), O2B, pb))
        o1b.wait_recv()
        o2.append(rcopy(ostrip(pb, 1), O2A, pa))
        for r in o2:
            r.wait_recv()

        for r in pending:
            r.wait_send()

    return pl.pallas_call(
        body,
        out_shape=jax.ShapeDtypeStruct((N_DEV * B_LOC, D), BF16),
        in_specs=[pl.BlockSpec(memory_space=pltpu.VMEM)] * 7,
        out_specs=pl.BlockSpec(memory_space=pltpu.VMEM),
        scratch_shapes=[
            pltpu.VMEM((3, D, H), BF16),
            pltpu.VMEM((3, H, D), BF16),
            pltpu.SemaphoreType.DMA((16,)),
            pltpu.SemaphoreType.DMA((16,)),
        ],
        compiler_params=pltpu.CompilerParams(collective_id=0),
    )(x, Win0, Wout0, Win1, Wout1, Win2, Wout2)


# device time: 45763 ns/iter; 1.0175x vs baseline; 1.0175x over previous
import os

import jax
import jax.numpy as jnp
from jax import lax
from jax.experimental import pallas as pl
from jax.experimental.pallas import tpu as pltpu

N_DEV = 4
B_LOC = 512
D = 256
H_LOC = 512
H = N_DEV * H_LOC
BF16 = jnp.bfloat16

SEM = {
    (0, "r1a"): 0, (0, "r1b"): 1, (0, "e2a"): 2, (0, "e2b"): 3,
    (1, "r1a"): 4, (1, "r1b"): 5, (1, "e2a"): 6, (1, "e2b"): 7,
    (2, "r1a"): 8, (2, "r1b"): 9, (2, "e2a"): 10, (2, "e2b"): 11,
    (2, "x2a"): 12, (2, "x2b"): 13,
}
O1A, O1B, O2A, O2B = 14, 15, 16, 17
N_SEM = 18
SEM[(0, "x2a")] = SEM[(0, "e2a")]
SEM[(0, "x2b")] = SEM[(0, "e2b")]
SEM[(1, "x2a")] = SEM[(1, "e2a")]
SEM[(1, "x2b")] = SEM[(1, "e2b")]

_SKIP_COMM = bool(int(os.environ.get("SKIP_COMM", "0")))
_SKIP_MM = bool(int(os.environ.get("SKIP_MM", "0")))


def kernel(x, Win0, Wout0, Win1, Wout1, Win2, Wout2):
    def body(x_ref, win0_ref, wout0_ref, win1_ref, wout1_ref, win2_ref,
             wout2_ref, out_ref, winb_ref, woutb_ref, send_sems, recv_sems):
        j = lax.axis_index("i")
        pa = jnp.bitwise_xor(j, 1)
        pb = jnp.bitwise_xor(j, 3)
        dg = jnp.bitwise_xor(j, 2)

        barrier = pltpu.get_barrier_semaphore()
        for nbr in (pa, pb):
            pl.semaphore_signal(barrier, inc=1, device_id=(nbr,),
                                device_id_type=pl.DeviceIdType.MESH)
        pl.semaphore_wait(barrier, 2)

        pending = []

        class _Dummy:
            def wait_recv(self):
                pass

            def wait_send(self):
                pass

        def rcopy(ref_expr, sem_idx, target):
            if _SKIP_COMM:
                return _Dummy()
            r = pltpu.make_async_remote_copy(
                src_ref=ref_expr,
                dst_ref=ref_expr,
                send_sem=send_sems.at[sem_idx],
                recv_sem=recv_sems.at[sem_idx],
                device_id=(target,),
                device_id_type=pl.DeviceIdType.MESH,
            )
            r.start()
            pending.append(r)
            return r

        def cols(k, half):
            return pl.ds(pl.multiple_of(k * H_LOC + 256 * half, 256), 256)

        def send_pair(k, l, half, sem_idx, target):
            return [rcopy(winb_ref.at[l, :, cols(k, half)], sem_idx, target),
                    rcopy(woutb_ref.at[l, cols(k, half), :], sem_idx, target)]

        in_refs = ((win0_ref, wout0_ref), (win1_ref, wout1_ref),
                   (win2_ref, wout2_ref))

        cur = x_ref[:, :].astype(BF16)
        r2_descs = {0: [], 1: []}
        l2_grp = {}

        def compute_layer(l, cur):
            for r in r2_descs[l]:
                r.wait_recv()
            if _SKIP_MM:
                return cur
            h = jnp.maximum(
                jnp.dot(cur, winb_ref[l, :, :],
                        preferred_element_type=jnp.float32),
                0.0,
            ).astype(BF16)
            return jnp.dot(h, woutb_ref[l, :, :],
                           preferred_element_type=jnp.float32).astype(BF16)

        for l, (win, wout) in enumerate(in_refs):
            myblk = pl.ds(pl.multiple_of(j * H_LOC, H_LOC), H_LOC)
            winb_ref[l, :, myblk] = win[:, :].astype(BF16)
            woutb_ref[l, myblk, :] = wout[:, :].astype(BF16)
            r1a = send_pair(j, l, 0, SEM[(l, "r1a")], pa)
            r1b = send_pair(j, l, 1, SEM[(l, "r1b")], pb)
            e2b = send_pair(j, l, 0, SEM[(l, "e2b")], pb)
            e2a = send_pair(j, l, 1, SEM[(l, "e2a")], pa)
            for r in r1a:
                r.wait_recv()
            x2b = send_pair(pa, l, 0, SEM[(l, "x2b")], pb)
            for r in r1b:
                r.wait_recv()
            x2a = send_pair(pb, l, 1, SEM[(l, "x2a")], pa)
            if l < 2:
                r2_descs[l] = e2b + e2a + x2b + x2a
            else:
                l2_grp["pa"] = e2a
                l2_grp["pb"] = e2b
                l2_grp["dg"] = x2a + x2b
            if l >= 1:
                cur = compute_layer(l - 1, cur)

        def block_partial(k, cur):
            if _SKIP_MM:
                return cur.astype(jnp.float32)
            wslice = winb_ref[2, :, pl.ds(pl.multiple_of(k * H_LOC, H_LOC),
                                          H_LOC)]
            h = jnp.maximum(
                jnp.dot(cur, wslice, preferred_element_type=jnp.float32),
                0.0,
            ).astype(BF16)
            return jnp.dot(
                h,
                woutb_ref[2, pl.ds(pl.multiple_of(k * H_LOC, H_LOC), H_LOC), :],
                preferred_element_type=jnp.float32)

        acc = block_partial(j, cur)
        for r in l2_grp["pa"]:
            r.wait_recv()
        acc = acc + block_partial(pa, cur)
        for r in l2_grp["pb"]:
            r.wait_recv()
        acc = acc + block_partial(pb, cur)
        for r in l2_grp["dg"]:
            r.wait_recv()
        acc = acc + block_partial(dg, cur)
        myrows = pl.ds(pl.multiple_of(j * B_LOC, B_LOC), B_LOC)
        out_ref[myrows, :] = acc.astype(BF16)

        def ostrip(k, half):
            return out_ref.at[
                pl.ds(pl.multiple_of(k * B_LOC + 256 * half, 256), 256), :]

        o1a = rcopy(ostrip(j, 0), O1A, pa)
        o1b = rcopy(ostrip(j, 1), O1B, pb)
        o2 = [rcopy(ostrip(j, 0), O2B, pb),
              rcopy(ostrip(j, 1), O2A, pa)]
        o1a.wait_recv()
        o2.append(rcopy(ostrip(pa, 0), O2B, pb))
        o1b.wait_recv()
        o2.append(rcopy(ostrip(pb, 1), O2A, pa))
        for r in o2:
            r.wait_recv()

        for r in pending:
            r.wait_send()

    return pl.pallas_call(
        body,
        out_shape=jax.ShapeDtypeStruct((N_DEV * B_LOC, D), BF16),
        in_specs=[pl.BlockSpec(memory_space=pltpu.VMEM)] * 7,
        out_specs=pl.BlockSpec(memory_space=pltpu.VMEM),
        scratch_shapes=[
            pltpu.VMEM((3, D, H), BF16),
            pltpu.VMEM((3, H, D), BF16),
            pltpu.SemaphoreType.DMA((N_SEM,)),
            pltpu.SemaphoreType.DMA((N_SEM,)),
        ],
        compiler_params=pltpu.CompilerParams(collective_id=0),
    )(x, Win0, Wout0, Win1, Wout1, Win2, Wout2)
